# baseline (speedup 1.0000x reference)
import math

import jax
import jax.numpy as jnp
from jax.experimental import pallas as pl
from jax.experimental.pallas import tpu as pltpu

_SIZE = 100000
_PAD = 0
_SV = 0.1 / (_SIZE - 2)
_CONF = 0.9
_C_ROW = (_SIZE - 2) * _SV * math.log(_SV) + _CONF * math.log(_CONF)

_N = 1024
_BC = 2048
_GRID = (_SIZE + _BC - 1) // _BC


def _tc_kernel(mask_ref, out_ref, tgt_ref, acc_ref, rowacc_ref, gacc_ref):
    j = pl.program_id(0)
    x = out_ref[...]
    mk = mask_ref[...].reshape(1, _BC)
    xm = jnp.where(mk != 0, x, 0.0)
    cols = j * _BC + jax.lax.broadcasted_iota(jnp.int32, x.shape, 1)
    t = tgt_ref[...]

    @pl.when(j == 0)
    def _init():
        rowacc_ref[...] = jnp.zeros_like(rowacc_ref)
        gacc_ref[...] = jnp.zeros_like(gacc_ref)

    rowacc_ref[...] += jnp.sum(xm, axis=1, keepdims=True)
    gacc_ref[...] += jnp.sum(jnp.where(cols == t, xm, 0.0), axis=1, keepdims=True)

    @pl.when(j == _GRID - 1)
    def _combine():
        g = gacc_ref[...]
        per_row = _C_ROW - _SV * (rowacc_ref[...] - g) - _CONF * g
        acc_ref[0, 0] = jnp.sum(jnp.where(t != _PAD, per_row, 0.0))


def kernel(output, target, normalize):
    tgt = target.astype(jnp.int32)

    cols = jnp.arange(_GRID * _BC, dtype=jnp.int32)
    mask = ((cols != _PAD) & (cols < _SIZE)).astype(jnp.int32)
    mask = mask.reshape(_GRID, 1, _BC)

    acc = pl.pallas_call(
        _tc_kernel,
        grid=(_GRID,),
        in_specs=[
            pl.BlockSpec((1, 1, _BC), lambda j: (j, 0, 0)),
            pl.BlockSpec((_N, _BC), lambda j: (0, j)),
            pl.BlockSpec((_N, 1), lambda j: (0, 0)),
        ],
        out_specs=pl.BlockSpec((1, 1), lambda j: (0, 0), memory_space=pltpu.SMEM),
        out_shape=jax.ShapeDtypeStruct((1, 1), jnp.float32),
        scratch_shapes=[
            pltpu.VMEM((_N, 1), jnp.float32),
            pltpu.VMEM((_N, 1), jnp.float32),
        ],
    )(mask, output, tgt)
    return acc[0, 0] / jnp.asarray(normalize, dtype=jnp.float32)


# speedup vs baseline: 1.0219x; 1.0219x over previous
import math

import jax
import jax.numpy as jnp
from jax.experimental import pallas as pl
from jax.experimental.pallas import tpu as pltpu

_SIZE = 100000
_PAD = 0
_SV = 0.1 / (_SIZE - 2)
_CONF = 0.9
_C_ROW = (_SIZE - 2) * _SV * math.log(_SV) + _CONF * math.log(_CONF)

_N = 1024
_NS = 4
_RS = _N // _NS
_BC = 4096
_GRID = (_SIZE + _BC - 1) // _BC


def _tc_kernel(mask_ref, x0, x1, x2, x3, tgt_ref, acc_ref, rowacc_ref, gacc_ref):
    j = pl.program_id(0)
    mk = mask_ref[...].reshape(1, _BC)
    t = tgt_ref[...]

    @pl.when(j == 0)
    def _init():
        rowacc_ref[...] = jnp.zeros_like(rowacc_ref)
        gacc_ref[...] = jnp.zeros_like(gacc_ref)

    for k, xr in enumerate((x0, x1, x2, x3)):
        x = xr[...]
        xm = jnp.where(mk != 0, x, 0.0)
        cols = j * _BC + jax.lax.broadcasted_iota(jnp.int32, x.shape, 1)
        tk = t[k * _RS:(k + 1) * _RS, :]
        rowacc_ref[k * _RS:(k + 1) * _RS, :] += jnp.sum(xm, axis=1, keepdims=True)
        gacc_ref[k * _RS:(k + 1) * _RS, :] += jnp.sum(
            jnp.where(cols == tk, xm, 0.0), axis=1, keepdims=True)

    @pl.when(j == _GRID - 1)
    def _combine():
        g = gacc_ref[...]
        per_row = _C_ROW - _SV * (rowacc_ref[...] - g) - _CONF * g
        acc_ref[0, 0] = jnp.sum(jnp.where(t != _PAD, per_row, 0.0))


def kernel(output, target, normalize):
    tgt = target.astype(jnp.int32)

    cols = jnp.arange(_GRID * _BC, dtype=jnp.int32)
    mask = ((cols != _PAD) & (cols < _SIZE)).astype(jnp.int32)
    mask = mask.reshape(_GRID, 1, _BC)

    acc = pl.pallas_call(
        _tc_kernel,
        grid=(_GRID,),
        in_specs=[pl.BlockSpec((1, 1, _BC), lambda j: (j, 0, 0))]
        + [pl.BlockSpec((_RS, _BC), lambda j, k=k: (k, j)) for k in range(_NS)]
        + [pl.BlockSpec((_N, 1), lambda j: (0, 0))],
        out_specs=pl.BlockSpec((1, 1), lambda j: (0, 0), memory_space=pltpu.SMEM),
        out_shape=jax.ShapeDtypeStruct((1, 1), jnp.float32),
        scratch_shapes=[
            pltpu.VMEM((_N, 1), jnp.float32),
            pltpu.VMEM((_N, 1), jnp.float32),
        ],
    )(mask, output, output, output, output, tgt)
    return acc[0, 0] / jnp.asarray(normalize, dtype=jnp.float32)
